# baseline (speedup 1.0000x reference)
import functools

import jax
import jax.numpy as jnp
from jax import lax
from jax.experimental import pallas as pl
from jax.experimental.pallas import tpu as pltpu

ROWS = 128
COLS = 100000
BLOCK_COLS = 2048
GRID = (COLS + BLOCK_COLS - 1) // BLOCK_COLS
BIG_IDX = 2**30


def _rotl(x, d):
    return (x << jnp.uint32(d)) | (x >> jnp.uint32(32 - d))


def _threefry_bits(p):
    ks0 = jnp.uint32(0)
    ks1 = jnp.uint32(42)
    ks2 = ks0 ^ ks1 ^ jnp.uint32(0x1BD11BDA)
    x0 = jnp.zeros_like(p) + ks0
    x1 = p + ks1
    rot = ((13, 15, 26, 6), (17, 29, 16, 24))
    inj = ((ks1, ks2), (ks2, ks0), (ks0, ks1), (ks1, ks2), (ks2, ks0))
    for i in range(5):
        for r in rot[i % 2]:
            x0 = x0 + x1
            x1 = _rotl(x1, r) ^ x0
        x0 = x0 + inj[i][0]
        x1 = x1 + inj[i][1] + jnp.uint32(i + 1)
    return x0 ^ x1


def _gumbel_from_bits(bits):
    m = bits >> jnp.uint32(9)
    f = jax.lax.bitcast_convert_type(m | jnp.uint32(0x3F800000), jnp.float32)
    f = f - jnp.float32(1.0)
    u = jnp.maximum(f, jnp.float32(1.1754943508222875e-38))
    w = -jnp.log(u)
    return -jnp.log(w)


def _sample_block_kernel(t_ref, x_ref, o_ref, vmax_ref, vidx_ref):
    step = pl.program_id(0)
    base = step * BLOCK_COLS

    @pl.when(step == 0)
    def _init():
        vmax_ref[...] = jnp.full((ROWS, 1), -jnp.inf, jnp.float32)
        vidx_ref[...] = jnp.zeros((ROWS, 1), jnp.int32)

    col = jnp.uint32(base) + lax.broadcasted_iota(
        jnp.uint32, (ROWS, BLOCK_COLS), 1)
    row = lax.broadcasted_iota(jnp.uint32, (ROWS, BLOCK_COLS), 0)
    p = row * jnp.uint32(COLS) + col

    g = _gumbel_from_bits(_threefry_bits(p))
    key = x_ref[...] / t_ref[0] + g
    key = jnp.where(col < jnp.uint32(COLS), key, -jnp.inf)

    bmax = jnp.max(key, axis=1, keepdims=True)
    colv = jnp.int32(base) + lax.broadcasted_iota(
        jnp.int32, (ROWS, BLOCK_COLS), 1)
    bidx = jnp.min(jnp.where(key == bmax, colv, BIG_IDX), axis=1, keepdims=True)

    better = bmax > vmax_ref[...]
    vmax_ref[...] = jnp.where(better, bmax, vmax_ref[...])
    vidx_ref[...] = jnp.where(better, bidx, vidx_ref[...])

    @pl.when(step == GRID - 1)
    def _fin():
        o_ref[...] = vidx_ref[...]


def kernel(logits, temperature):
    t = temperature.astype(jnp.float32)
    out = pl.pallas_call(
        _sample_block_kernel,
        grid=(GRID,),
        in_specs=[
            pl.BlockSpec(memory_space=pltpu.SMEM),
            pl.BlockSpec((ROWS, BLOCK_COLS), lambda i: (0, i)),
        ],
        out_specs=pl.BlockSpec((ROWS, 1), lambda i: (0, 0)),
        out_shape=jax.ShapeDtypeStruct((ROWS, 1), jnp.int32),
        scratch_shapes=[
            pltpu.VMEM((ROWS, 1), jnp.float32),
            pltpu.VMEM((ROWS, 1), jnp.int32),
        ],
    )(t, logits)
    return out
